# Initial kernel scaffold; baseline (speedup 1.0000x reference)
import jax
import jax.numpy as jnp
from jax.experimental import pallas as pl


def kernel(xyz1, xyz2, points1, points2, W1, b1, g1, be1, W2, b2, g2, be2):
    raise NotImplementedError("write your pallas kernel here")


# speedup vs baseline: 19.0738x; 19.0738x over previous
import functools

import jax
import jax.numpy as jnp
from jax.experimental import pallas as pl


def _stage1_body(xyz1_ref, xyz2_ref, p1_ref, p2_ref, w1_ref, y1_ref, st_ref,
                 *, S):
    q = xyz1_ref[0]
    k = xyz2_ref[0]
    TN = q.shape[0]
    qk = jax.lax.dot_general(q, k, (((1,), (1,)), ((), ())),
                             preferred_element_type=jnp.float32)
    qn = jnp.sum(q * q, axis=1, keepdims=True)
    kn = jnp.sum(k * k, axis=1, keepdims=True)
    d = -2.0 * qk + qn + kn.reshape(1, S)
    iota = jax.lax.broadcasted_iota(jnp.int32, (TN, S), 1)
    ohs, vals = [], []
    for _ in range(3):
        m = jnp.min(d, axis=1, keepdims=True)
        eq = d == m
        sel = jnp.min(jnp.where(eq, iota, S), axis=1, keepdims=True)
        oh = iota == sel
        ohs.append(oh)
        vals.append(m)
        d = jnp.where(oh, jnp.float32(jnp.inf), d)
    r = [1.0 / (v + 1e-8) for v in vals]
    inv = 1.0 / (r[0] + r[1] + r[2])
    wd = (jnp.where(ohs[0], r[0] * inv, 0.0)
          + jnp.where(ohs[1], r[1] * inv, 0.0)
          + jnp.where(ohs[2], r[2] * inv, 0.0))
    p2 = p2_ref[0]
    interp = jax.lax.dot_general(p2, wd, (((1,), (1,)), ((), ())),
                                 preferred_element_type=jnp.float32)
    p1 = p1_ref[0]
    C1 = p1.shape[0]
    y = (jnp.dot(w1_ref[:, :C1], p1, preferred_element_type=jnp.float32)
         + jnp.dot(w1_ref[:, C1:], interp,
                   preferred_element_type=jnp.float32))
    y1_ref[0] = y
    s = jnp.sum(y, axis=1)
    sq = jnp.sum(y * y, axis=1)
    new = jnp.concatenate([s[None, :], sq[None, :]], axis=0)
    first = jnp.logical_and(pl.program_id(0) == 0, pl.program_id(1) == 0)

    @pl.when(first)
    def _():
        st_ref[...] = new

    @pl.when(jnp.logical_not(first))
    def _():
        st_ref[...] = st_ref[...] + new


def _stage2_body(y1_ref, st_ref, g1_ref, be1_ref, w2_ref, y2_ref, st2_ref,
                 *, NB):
    mean = st_ref[0, :] * (1.0 / NB)
    var = st_ref[1, :] * (1.0 / NB) - mean * mean
    scale = g1_ref[0, :] * jax.lax.rsqrt(var + 1e-5)
    shift = be1_ref[0, :] - mean * scale
    z = jnp.maximum(y1_ref[0] * scale[:, None] + shift[:, None], 0.0)
    y2 = jnp.dot(w2_ref[...], z, preferred_element_type=jnp.float32)
    y2_ref[0] = y2
    s = jnp.sum(y2, axis=1)
    sq = jnp.sum(y2 * y2, axis=1)
    new = jnp.concatenate([s[None, :], sq[None, :]], axis=0)
    first = jnp.logical_and(pl.program_id(0) == 0, pl.program_id(1) == 0)

    @pl.when(first)
    def _():
        st2_ref[...] = new

    @pl.when(jnp.logical_not(first))
    def _():
        st2_ref[...] = st2_ref[...] + new


def _stage3_body(y2_ref, st2_ref, g2_ref, be2_ref, o_ref, *, NB):
    mean = st2_ref[0, :] * (1.0 / NB)
    var = st2_ref[1, :] * (1.0 / NB) - mean * mean
    scale = g2_ref[0, :] * jax.lax.rsqrt(var + 1e-5)
    shift = be2_ref[0, :] - mean * scale
    o_ref[0] = jnp.maximum(y2_ref[0] * scale[:, None] + shift[:, None], 0.0)


@jax.jit
def kernel(xyz1, xyz2, points1, points2, W1, b1, g1, be1, W2, b2, g2, be2):
    B, N, _ = xyz1.shape
    S = xyz2.shape[1]
    C1 = points1.shape[1]
    C2 = points2.shape[1]
    H1 = W1.shape[0]
    H2 = W2.shape[0]
    NB = B * N
    TN = 512

    xyz1p = jnp.pad(xyz1, ((0, 0), (0, 0), (0, 5)))
    xyz2p = jnp.pad(xyz2, ((0, 0), (0, 0), (0, 5)))

    y1, st1 = pl.pallas_call(
        functools.partial(_stage1_body, S=S),
        grid=(B, N // TN),
        in_specs=[
            pl.BlockSpec((1, TN, 8), lambda b, n: (b, n, 0)),
            pl.BlockSpec((1, S, 8), lambda b, n: (b, 0, 0)),
            pl.BlockSpec((1, C1, TN), lambda b, n: (b, 0, n)),
            pl.BlockSpec((1, C2, S), lambda b, n: (b, 0, 0)),
            pl.BlockSpec((H1, C1 + C2), lambda b, n: (0, 0)),
        ],
        out_specs=[
            pl.BlockSpec((1, H1, TN), lambda b, n: (b, 0, n)),
            pl.BlockSpec((2, H1), lambda b, n: (0, 0)),
        ],
        out_shape=[
            jax.ShapeDtypeStruct((B, H1, N), jnp.float32),
            jax.ShapeDtypeStruct((2, H1), jnp.float32),
        ],
    )(xyz1p, xyz2p, points1, points2, W1)

    y2, st2 = pl.pallas_call(
        functools.partial(_stage2_body, NB=NB),
        grid=(B, N // TN),
        in_specs=[
            pl.BlockSpec((1, H1, TN), lambda b, n: (b, 0, n)),
            pl.BlockSpec((2, H1), lambda b, n: (0, 0)),
            pl.BlockSpec((1, H1), lambda b, n: (0, 0)),
            pl.BlockSpec((1, H1), lambda b, n: (0, 0)),
            pl.BlockSpec((H2, H1), lambda b, n: (0, 0)),
        ],
        out_specs=[
            pl.BlockSpec((1, H2, TN), lambda b, n: (b, 0, n)),
            pl.BlockSpec((2, H2), lambda b, n: (0, 0)),
        ],
        out_shape=[
            jax.ShapeDtypeStruct((B, H2, N), jnp.float32),
            jax.ShapeDtypeStruct((2, H2), jnp.float32),
        ],
    )(y1, st1, g1.reshape(1, H1), be1.reshape(1, H1), W2)

    out = pl.pallas_call(
        functools.partial(_stage3_body, NB=NB),
        grid=(B, N // TN),
        in_specs=[
            pl.BlockSpec((1, H2, TN), lambda b, n: (b, 0, n)),
            pl.BlockSpec((2, H2), lambda b, n: (0, 0)),
            pl.BlockSpec((1, H2), lambda b, n: (0, 0)),
            pl.BlockSpec((1, H2), lambda b, n: (0, 0)),
        ],
        out_specs=pl.BlockSpec((1, H2, TN), lambda b, n: (b, 0, n)),
        out_shape=jax.ShapeDtypeStruct((B, H2, N), jnp.float32),
    )(y2, st2, g2.reshape(1, H2), be2.reshape(1, H2))
    return out
